# Initial kernel scaffold; baseline (speedup 1.0000x reference)
import jax
import jax.numpy as jnp
from jax.experimental import pallas as pl


def kernel(x, edge_index, batch, W1, b1, W2, b2, W3, b3, g1, be1, g2, be2, g3, be3, W_ih, W_hh, b_ih, b_hh, W_out, b_out):
    raise NotImplementedError("write your pallas kernel here")


# speedup vs baseline: 13.1129x; 13.1129x over previous
import functools

import jax
import jax.numpy as jnp
from jax import lax
from jax.experimental import pallas as pl
from jax.experimental.pallas import tpu as pltpu
from jax.experimental.pallas import tpu_sc as plsc

N = 10000
NP = 10240
E = 320000
F_IN = 128
H = 128
G = 64
OUT_DIM = 64

NC = 2
NS = 16
NW = NC * NS
C = 128
NCH = E // C
NCH_FULL = NCH // NW
NCH_REM = NCH % NW
RPT = NP // NS

_MESH = plsc.VectorSubcoreMesh(core_axis_name="c", subcore_axis_name="s")
_F32 = jnp.float32
_HIGH = lax.Precision.HIGHEST


def _dot(a, b):
    return jnp.dot(a, b, precision=_HIGH, preferred_element_type=_F32)


def _dot_t(a, b):
    return lax.dot_general(a, b, (((1,), (1,)), ((), ())),
                           precision=_HIGH, preferred_element_type=_F32)


@functools.partial(
    pl.kernel,
    out_type=jax.ShapeDtypeStruct((NC, NP, H), _F32),
    mesh=_MESH,
    scratch_types=[
        pltpu.VMEM((C,), jnp.int32),
        pltpu.VMEM((C,), jnp.int32),
        pltpu.VMEM((C, H), _F32),
        pltpu.VMEM_SHARED((NP, H), _F32),
        pltpu.SemaphoreType.DMA,
    ],
)
def _edge_agg(y_hbm, src_hbm, dst_hbm, zero_hbm, out_hbm,
              src_v, dst_v, rows_v, acc_sh, sem):
    cid = lax.axis_index("c")
    sid = lax.axis_index("s")
    wid = cid * NS + sid
    r0 = sid * RPT

    pltpu.sync_copy(zero_hbm.at[pl.ds(r0, RPT)], acc_sh.at[pl.ds(r0, RPT)])
    plsc.subcore_barrier()

    def _chunk(ch):
        base = ch * C
        pltpu.sync_copy(src_hbm.at[pl.ds(base, C)], src_v)
        pltpu.sync_copy(dst_hbm.at[pl.ds(base, C)], dst_v)
        pltpu.async_copy(y_hbm.at[src_v], rows_v, sem).wait()
        pltpu.sync_copy(rows_v, acc_sh.at[dst_v], add=True)

    @pl.loop(0, NCH_FULL)
    def _(t):
        _chunk(wid + NW * t)

    @pl.when(wid < NCH_REM)
    def _():
        _chunk(wid + NW * NCH_FULL)

    plsc.subcore_barrier()
    pltpu.sync_copy(acc_sh.at[pl.ds(r0, RPT)], out_hbm.at[cid, pl.ds(r0, RPT)])


@functools.partial(
    pl.kernel,
    out_type=jax.ShapeDtypeStruct((NC, NP, H), _F32),
    mesh=_MESH,
    scratch_types=[
        pltpu.VMEM((C,), jnp.int32),
        pltpu.VMEM((C, H), _F32),
        pltpu.VMEM_SHARED((NP, H), _F32),
    ],
)
def _deg_count(dst_hbm, ones_hbm, zero_hbm, out_hbm, dst_v, ones_v, acc_sh):
    cid = lax.axis_index("c")
    sid = lax.axis_index("s")
    wid = cid * NS + sid
    r0 = sid * RPT

    pltpu.sync_copy(zero_hbm.at[pl.ds(r0, RPT)], acc_sh.at[pl.ds(r0, RPT)])
    pltpu.sync_copy(ones_hbm, ones_v)
    plsc.subcore_barrier()

    def _chunk(ch):
        base = ch * C
        pltpu.sync_copy(dst_hbm.at[pl.ds(base, C)], dst_v)
        pltpu.sync_copy(ones_v, acc_sh.at[dst_v], add=True)

    @pl.loop(0, NCH_FULL)
    def _(t):
        _chunk(wid + NW * t)

    @pl.when(wid < NCH_REM)
    def _():
        _chunk(wid + NW * NCH_FULL)

    plsc.subcore_barrier()
    pltpu.sync_copy(acc_sh.at[pl.ds(r0, RPT)], out_hbm.at[cid, pl.ds(r0, RPT)])


def _prep0_body(degp_ref, x_ref, w_ref, dinv_ref, y_ref):
    deg = degp_ref[0, :, 0:1] + degp_ref[1, :, 0:1] + 1.0
    dinv = lax.rsqrt(deg)
    dinv_ref[...] = dinv
    y_ref[...] = dinv * _dot(x_ref[...], w_ref[...])


def _prep0(degp, x_pad, w1):
    return pl.pallas_call(
        _prep0_body,
        out_shape=(jax.ShapeDtypeStruct((NP, 1), _F32),
                   jax.ShapeDtypeStruct((NP, H), _F32)),
    )(degp, x_pad, w1)


def _bn_relu(agg, g, be):
    mask = (lax.broadcasted_iota(jnp.int32, (NP, 1), 0) < N).astype(_F32)
    m = jnp.sum(agg, axis=0, keepdims=True) * (1.0 / N)
    d = agg - m
    dm = d * mask
    v = jnp.sum(dm * dm, axis=0, keepdims=True) * (1.0 / N)
    return jnp.maximum(d * lax.rsqrt(v + 1e-5) * g + be, 0.0) * mask


def _post_prep_body(p_ref, y_ref, dinv_ref, g_ref, be_ref, wn_ref, out_ref):
    agg = dinv_ref[...] * (p_ref[0] + p_ref[1] + y_ref[...])
    hb = _bn_relu(agg, g_ref[...], be_ref[...])
    out_ref[...] = dinv_ref[...] * _dot(hb, wn_ref[...])


def _post_prep(p, y, dinv, g, be, wn):
    return pl.pallas_call(
        _post_prep_body,
        out_shape=jax.ShapeDtypeStruct((NP, H), _F32),
    )(p, y, dinv, g, be, wn)


def _final_body(p_ref, y_ref, dinv_ref, g_ref, be_ref, batch_ref,
                wih_ref, bih_ref, bhh_ref, wout_ref, bout_ref, out_ref):
    agg = dinv_ref[...] * (p_ref[0] + p_ref[1] + y_ref[...])
    h3 = _bn_relu(agg, g_ref[...], be_ref[...])

    gid = lax.broadcasted_iota(jnp.int32, (G, NP), 0)
    onehot = (batch_ref[...] == gid).astype(_F32)
    psum = _dot(onehot, h3)
    cnt = _dot(onehot, jnp.ones((NP, 1), _F32))
    pooled = psum / jnp.maximum(cnt, 1.0)

    gates = _dot_t(pooled, wih_ref[...]) + bih_ref[...] + bhh_ref[...]
    i = jax.nn.sigmoid(gates[:, :H])
    f = jax.nn.sigmoid(gates[:, H:2 * H])
    gg = jnp.tanh(gates[:, 2 * H:3 * H])
    o = jax.nn.sigmoid(gates[:, 3 * H:])
    del f
    hn = o * jnp.tanh(i * gg)
    out_ref[...] = _dot_t(hn, wout_ref[...]) + bout_ref[...]


def _final(p, y, dinv, g, be, batch_row, w_ih, b_ih, b_hh, w_out, b_out):
    return pl.pallas_call(
        _final_body,
        out_shape=jax.ShapeDtypeStruct((G, OUT_DIM), _F32),
    )(p, y, dinv, g, be, batch_row, w_ih, b_ih, b_hh, w_out, b_out)


def kernel(x, edge_index, batch, W1, b1, W2, b2, W3, b3,
           g1, be1, g2, be2, g3, be3,
           W_ih, W_hh, b_ih, b_hh, W_out, b_out):
    src = edge_index[0]
    dst = edge_index[1]
    zero_nh = jnp.zeros((NP, H), _F32)
    ones_ch = jnp.ones((C, H), _F32)
    x_pad = jnp.concatenate([x, jnp.zeros((NP - N, F_IN), _F32)], axis=0)
    batch_pad = jnp.concatenate(
        [batch, jnp.full((NP - N,), -1, batch.dtype)]).reshape(1, NP)

    degp = _deg_count(dst, ones_ch, zero_nh)
    dinv, y = _prep0(degp, x_pad, W1)

    for g_, be_, wn in ((g1, be1, W2), (g2, be2, W3)):
        p = _edge_agg(y, src, dst, zero_nh)
        y = _post_prep(p, y, dinv, g_.reshape(1, H), be_.reshape(1, H), wn)

    p = _edge_agg(y, src, dst, zero_nh)
    return _final(p, y, dinv, g3.reshape(1, H), be3.reshape(1, H),
                  batch_pad, W_ih, b_ih.reshape(1, 4 * H),
                  b_hh.reshape(1, 4 * H), W_out, b_out.reshape(1, OUT_DIM))
